# Initial kernel scaffold; baseline (speedup 1.0000x reference)
import jax
import jax.numpy as jnp
from jax.experimental import pallas as pl


def kernel(x, Wr, W13, W2, g_norm):
    raise NotImplementedError("write your pallas kernel here")


# speedup vs baseline: 1.8584x; 1.8584x over previous
import functools

import jax
import jax.numpy as jnp
from jax.experimental import pallas as pl
from jax.experimental.pallas import tpu as pltpu


def _moe_body(x_ref, wr_ref, w13g_ref, w13u_ref, w2_ref, g_ref,
              o_ref, w_all, zb, acc, *, T, D, E, HC):
    e = pl.program_id(0)
    hc = pl.program_id(1)

    @pl.when((e == 0) & (hc == 0))
    def _router():
        logits = jax.lax.dot_general(
            x_ref[...], wr_ref[...], (((1,), (1,)), ((), ())),
            precision=jax.lax.Precision.DEFAULT,
            preferred_element_type=jnp.float32)
        iota = jax.lax.broadcasted_iota(jnp.int32, (T, E), 1)
        i1 = jnp.argmax(logits, axis=1)[:, None]
        m1 = jnp.max(logits, axis=1, keepdims=True)
        l2 = jnp.where(iota == i1, -jnp.inf, logits)
        i2 = jnp.argmax(l2, axis=1)[:, None]
        m2 = jnp.max(l2, axis=1, keepdims=True)
        g0 = jax.nn.sigmoid(m1 - m2)
        g1 = 1.0 - g0
        w_all[...] = (jnp.where(iota == i1, g0, 0.0)
                      + jnp.where(iota == i2, g1, 0.0))

    @pl.when(hc == 0)
    def _norm():
        z = 2.0 * x_ref[...]
        var = jnp.mean(z * z, axis=1, keepdims=True)
        zn = z * jax.lax.rsqrt(var + 1e-6) * g_ref[0]
        zb[...] = zn.astype(jnp.bfloat16)

    @pl.when((e == 0) & (hc == 0))
    def _init():
        acc[...] = jnp.zeros_like(acc)

    hg = jax.lax.dot_general(
        zb[...], w13g_ref[0].astype(jnp.bfloat16), (((1,), (1,)), ((), ())),
        preferred_element_type=jnp.float32)
    hu = jax.lax.dot_general(
        zb[...], w13u_ref[0].astype(jnp.bfloat16), (((1,), (1,)), ((), ())),
        preferred_element_type=jnp.float32)
    sw = (hg * jax.nn.sigmoid(hg) * hu).astype(jnp.bfloat16)
    proj = jax.lax.dot_general(
        sw, w2_ref[0].astype(jnp.bfloat16), (((1,), (1,)), ((), ())),
        preferred_element_type=jnp.float32)

    iota = jax.lax.broadcasted_iota(jnp.int32, (T, E), 1)
    w_e = jnp.sum(w_all[...] * (iota == e).astype(jnp.float32),
                  axis=1, keepdims=True)
    acc[...] += proj * w_e

    @pl.when((e == E - 1) & (hc == HC - 1))
    def _final():
        o_ref[...] = x_ref[...] + acc[...]


def _build_dense(T, D, E, H, HC, interpret=False):
    CH = H // HC
    body = functools.partial(_moe_body, T=T, D=D, E=E, HC=HC)
    return pl.pallas_call(
        body,
        grid=(E, HC),
        in_specs=[
            pl.BlockSpec((T, D), lambda e, hc: (0, 0)),
            pl.BlockSpec((E, D), lambda e, hc: (0, 0)),
            pl.BlockSpec((1, CH, D), lambda e, hc: (e, hc, 0)),
            pl.BlockSpec((1, CH, D), lambda e, hc: (e, hc + HC, 0)),
            pl.BlockSpec((1, D, CH), lambda e, hc: (e, 0, hc)),
            pl.BlockSpec((1, 1, D), lambda e, hc: (e, 0, 0)),
        ],
        out_specs=pl.BlockSpec((T, D), lambda e, hc: (0, 0)),
        out_shape=jax.ShapeDtypeStruct((T, D), jnp.float32),
        scratch_shapes=[
            pltpu.VMEM((T, E), jnp.float32),
            pltpu.VMEM((T, D), jnp.bfloat16),
            pltpu.VMEM((T, D), jnp.float32),
        ],
        compiler_params=pltpu.CompilerParams(
            dimension_semantics=("arbitrary", "arbitrary")),
        interpret=interpret,
    )


def kernel(x, Wr, W13, W2, g_norm):
    T, D = x.shape
    E = Wr.shape[0]
    H = W2.shape[2]
    call = _build_dense(T, D, E, H, HC=4)
    return call(x, Wr, W13, W13, W2, g_norm.reshape(E, 1, D))
